# baseline (speedup 1.0000x reference)
import functools

import jax
import jax.numpy as jnp
from jax import lax
from jax.experimental import pallas as pl
from jax.experimental.pallas import tpu as pltpu
from jax.experimental.pallas import tpu_sc as plsc

NSLOT = 10
NFEAT = 11
XCOL = 32
IDX_MINOR = 128


@functools.cache
def _build(B, V, D, DOUT):
    NC, NS = 2, 16
    NW = NC * NS
    BPW = B // NW
    CC = 64
    NCH = BPW // CC
    NP = NCH // 2
    NG = (CC * NSLOT) // IDX_MINOR
    IPW = (BPW * NSLOT) // IDX_MINOR
    ND = D // 16

    mesh = plsc.VectorSubcoreMesh(core_axis_name="c", subcore_axis_name="s")

    @functools.partial(
        pl.kernel,
        mesh=mesh,
        compiler_params=pltpu.CompilerParams(
            use_tc_tiling_on_sc=False, needs_layout_passes=False),
        out_type=jax.ShapeDtypeStruct((B, DOUT), jnp.float32),
        scratch_types=[
            pltpu.VMEM((BPW, XCOL), jnp.float32),
            pltpu.VMEM((IPW, IDX_MINOR), jnp.int32),
            pltpu.VMEM((2, CC * NSLOT, D), jnp.float32),
            pltpu.VMEM((2, CC, DOUT), jnp.float32),
            pltpu.VMEM((NFEAT, D), jnp.float32),
            pltpu.SemaphoreType.DMA,
            pltpu.SemaphoreType.DMA,
            pltpu.SemaphoreType.DMA,
            pltpu.SemaphoreType.DMA,
        ],
    )
    def enc(x_hbm, table_hbm, wl_hbm, out_hbm,
            x_v, idx_v, rows_v, out_v, wl_v, gsem0, gsem1, osem0, osem1):
        wid = lax.axis_index("s") * NC + lax.axis_index("c")
        base = wid * BPW
        gsems = [gsem0, gsem1]
        osems = [osem0, osem1]
        pltpu.sync_copy(x_hbm.at[pl.ds(base, BPW)], x_v)
        pltpu.sync_copy(wl_hbm, wl_v)

        lane = lax.iota(jnp.int32, 16)

        def extract_body(g, carry):
            for h in range(IDX_MINOR // 16):
                p = g * IDX_MINOR + h * 16 + lane
                r = p // NSLOT
                c = 2 * (p - NSLOT * r)
                vals = plsc.load_gather(x_v, [r, c])
                idx_v[g, pl.ds(h * 16, 16)] = vals.astype(jnp.int32)
            return carry

        lax.fori_loop(0, IPW, extract_body, 0, unroll=2)

        wcol = 1 + 2 * lane
        xcol = jnp.minimum(20 + lane, XCOL - 1)

        def gather_descs(ci, buf):
            return [
                pltpu.make_async_copy(
                    table_hbm.at[idx_v.at[ci * NG + g]],
                    rows_v.at[buf].at[pl.ds(g * IDX_MINOR, IDX_MINOR)],
                    gsems[buf],
                )
                for g in range(NG)
            ]

        def fire_gathers(ci, buf):
            for d in gather_descs(ci, buf):
                d.start()

        def out_desc(ci, buf):
            return pltpu.make_async_copy(
                out_v.at[buf], out_hbm.at[pl.ds(base + ci * CC, CC)], osems[buf])

        def compute_chunk(ci, buf):
            def row_body(i, carry):
                row = ci * CC + i
                rsplat = jnp.full((16,), row, jnp.int32)
                wrow = plsc.load_gather(x_v, [rsplat, wcol])
                xrow = plsc.load_gather(x_v, [rsplat, xcol])
                r0 = i * NSLOT
                w0 = wrow[0]
                accs = [w0 * rows_v[buf, r0, pl.ds(d * 16, 16)] for d in range(ND)]
                for j in range(1, NSLOT):
                    wgt = wrow[j]
                    r = r0 + j
                    for d in range(ND):
                        accs[d] = accs[d] + wgt * rows_v[buf, r, pl.ds(d * 16, 16)]
                x0 = xrow[0]
                lins = [x0 * wl_v[0, pl.ds(d * 16, 16)] for d in range(ND)]
                for k in range(1, NFEAT):
                    xk = xrow[k]
                    for d in range(ND):
                        lins[d] = lins[d] + xk * wl_v[k, pl.ds(d * 16, 16)]
                for d in range(ND):
                    out_v[buf, i, pl.ds(d * 16, 16)] = accs[d]
                    out_v[buf, i, pl.ds(D + d * 16, 16)] = lins[d]
                return carry

            lax.fori_loop(0, CC, row_body, 0, unroll=4)

        fire_gathers(0, 0)

        def pair_body(cp, carry):
            c0 = 2 * cp
            c1 = c0 + 1
            fire_gathers(c1, 1)
            for d in gather_descs(c0, 0):
                d.wait()

            @pl.when(cp >= 1)
            def _():
                out_desc(c0 - 2, 0).wait()

            compute_chunk(c0, 0)
            out_desc(c0, 0).start()

            @pl.when(cp < NP - 1)
            def _():
                fire_gathers(c0 + 2, 0)

            for d in gather_descs(c1, 1):
                d.wait()

            @pl.when(cp >= 1)
            def _():
                out_desc(c1 - 2, 1).wait()

            compute_chunk(c1, 1)
            out_desc(c1, 1).start()
            return carry

        lax.fori_loop(0, NP, pair_body, 0)
        out_desc(NCH - 2, 0).wait()
        out_desc(NCH - 1, 1).wait()

    return enc


def kernel(x, table, W_lin):
    B = x.shape[0]
    V, D = table.shape
    DOUT = D + W_lin.shape[0]
    xp = jnp.pad(x, ((0, 0), (0, XCOL - x.shape[1])))
    w_pre = (W_lin.T * (1.0 / 255.0)).astype(jnp.float32)
    enc = _build(B, V, D, DOUT)
    return enc(xp, table, w_pre)


# speedup vs baseline: 1.0229x; 1.0229x over previous
import functools

import jax
import jax.numpy as jnp
from jax import lax
from jax.experimental import pallas as pl
from jax.experimental.pallas import tpu as pltpu
from jax.experimental.pallas import tpu_sc as plsc

NSLOT = 10
NFEAT = 11
XCOL = 32
TROW = 128
IDX_MINOR = 64


@functools.cache
def _build(B, V, D, DOUT):
    NC, NS = 2, 16
    NW = NC * NS
    BPW = B // NW
    CC = 32
    NCH = BPW // CC
    NP = NCH // 2
    NG = (CC * NSLOT) // IDX_MINOR
    IPW = (BPW * NSLOT) // IDX_MINOR
    ND = D // 16

    mesh = plsc.VectorSubcoreMesh(core_axis_name="c", subcore_axis_name="s")

    @functools.partial(
        pl.kernel,
        mesh=mesh,
        compiler_params=pltpu.CompilerParams(
            use_tc_tiling_on_sc=False, needs_layout_passes=False),
        out_type=jax.ShapeDtypeStruct((B, DOUT), jnp.float32),
        scratch_types=[
            pltpu.VMEM((BPW, XCOL), jnp.float32),
            pltpu.VMEM((IPW, IDX_MINOR), jnp.int32),
            pltpu.VMEM((2, CC * NSLOT, TROW), jnp.float32),
            pltpu.VMEM((2, CC, DOUT), jnp.float32),
            pltpu.VMEM((NFEAT, D), jnp.float32),
            pltpu.SemaphoreType.DMA,
            pltpu.SemaphoreType.DMA,
            pltpu.SemaphoreType.DMA,
            pltpu.SemaphoreType.DMA,
        ],
    )
    def enc(x_hbm, table_hbm, wl_hbm, out_hbm,
            x_v, idx_v, rows_v, out_v, wl_v, gsem0, gsem1, osem0, osem1):
        wid = lax.axis_index("s") * NC + lax.axis_index("c")
        base = wid * BPW
        gsems = [gsem0, gsem1]
        osems = [osem0, osem1]
        pltpu.sync_copy(x_hbm.at[pl.ds(base, BPW)], x_v)
        pltpu.sync_copy(wl_hbm, wl_v)

        lane = lax.iota(jnp.int32, 16)

        def extract_body(g, carry):
            for h in range(IDX_MINOR // 16):
                p = g * IDX_MINOR + h * 16 + lane
                r = p // NSLOT
                c = 2 * (p - NSLOT * r)
                vals = plsc.load_gather(x_v, [r, c])
                idx_v[g, pl.ds(h * 16, 16)] = vals.astype(jnp.int32)
            return carry

        lax.fori_loop(0, IPW, extract_body, 0, unroll=2)

        wcol = 1 + 2 * lane
        xcol = jnp.minimum(20 + lane, XCOL - 1)

        def gather_descs(ci, buf):
            return [
                pltpu.make_async_copy(
                    table_hbm.at[idx_v.at[ci * NG + g]],
                    rows_v.at[buf].at[pl.ds(g * IDX_MINOR, IDX_MINOR)],
                    gsems[buf],
                )
                for g in range(NG)
            ]

        def fire_gathers(ci, buf):
            for d in gather_descs(ci, buf):
                d.start()

        def out_desc(ci, buf):
            return pltpu.make_async_copy(
                out_v.at[buf], out_hbm.at[pl.ds(base + ci * CC, CC)], osems[buf])

        def compute_chunk(ci, buf):
            def row_body(i, carry):
                row = ci * CC + i
                rsplat = jnp.full((16,), row, jnp.int32)
                wrow = plsc.load_gather(x_v, [rsplat, wcol])
                xrow = plsc.load_gather(x_v, [rsplat, xcol])
                r0 = i * NSLOT
                w0 = wrow[0]
                accs = [w0 * rows_v[buf, r0, pl.ds(d * 16, 16)] for d in range(ND)]
                for j in range(1, NSLOT):
                    wgt = wrow[j]
                    r = r0 + j
                    for d in range(ND):
                        accs[d] = accs[d] + wgt * rows_v[buf, r, pl.ds(d * 16, 16)]
                x0 = xrow[0]
                lins = [x0 * wl_v[0, pl.ds(d * 16, 16)] for d in range(ND)]
                for k in range(1, NFEAT):
                    xk = xrow[k]
                    for d in range(ND):
                        lins[d] = lins[d] + xk * wl_v[k, pl.ds(d * 16, 16)]
                for d in range(ND):
                    out_v[buf, i, pl.ds(d * 16, 16)] = accs[d]
                    out_v[buf, i, pl.ds(D + d * 16, 16)] = lins[d]
                return carry

            lax.fori_loop(0, CC, row_body, 0, unroll=2)

        fire_gathers(0, 0)

        def pair_body(cp, carry):
            c0 = 2 * cp
            c1 = c0 + 1
            fire_gathers(c1, 1)
            for d in gather_descs(c0, 0):
                d.wait()

            @pl.when(cp >= 1)
            def _():
                out_desc(c0 - 2, 0).wait()

            compute_chunk(c0, 0)
            out_desc(c0, 0).start()

            @pl.when(cp < NP - 1)
            def _():
                fire_gathers(c0 + 2, 0)

            for d in gather_descs(c1, 1):
                d.wait()

            @pl.when(cp >= 1)
            def _():
                out_desc(c1 - 2, 1).wait()

            compute_chunk(c1, 1)
            out_desc(c1, 1).start()
            return carry

        lax.fori_loop(0, NP, pair_body, 0)
        out_desc(NCH - 2, 0).wait()
        out_desc(NCH - 1, 1).wait()

    return enc


def kernel(x, table, W_lin):
    B = x.shape[0]
    V, D = table.shape
    DOUT = D + W_lin.shape[0]
    xp = jnp.pad(x, ((0, 0), (0, XCOL - x.shape[1])))
    tp = jnp.pad(table, ((0, 0), (0, TROW - D)))
    w_pre = (W_lin.T * (1.0 / 255.0)).astype(jnp.float32)
    enc = _build(B, V, D, DOUT)
    return enc(xp, tp, w_pre)
